# baseline (speedup 1.0000x reference)
import functools

import jax
import jax.numpy as jnp
from jax import lax
from jax.experimental import pallas as pl
from jax.experimental.pallas import tpu as pltpu
from jax.experimental.pallas import tpu_sc as plsc

N = 10000
Np = 10240
E = 320000
D = 128
B = 512
NC, NS, L = 2, 16, 16
NW = NC * NS

EPW = E // NW
EC = 40
ECN = EPW // EC
DEC = 80
DECN = EPW // DEC
RPS = Np // NS
RZC = 160
PC = 40
PCN = (N + PC - 1) // PC
PCI = (PCN + NW - 1) // NW

_MESH = plsc.VectorSubcoreMesh(
    core_axis_name="c", subcore_axis_name="s", num_cores=NC, num_subcores=NS
)

_f32 = jnp.float32


def _fill_rows(buf, nrows, ncols, value):
    vec = jnp.full((L,), value, _f32)

    def body(i, carry):
        for k in range(ncols // L):
            buf[i, pl.ds(k * L, L)] = vec
        return carry

    lax.fori_loop(0, nrows, body, 0)


NDSEM = 5


def _deg_body(dst3_hbm, degp_hbm, dacc, dstv, onesb, zb, ssems):
    c = lax.axis_index("c")
    s = lax.axis_index("s")
    wid = c * NS + s

    _fill_rows(onesb, DEC, L, 1.0)
    _fill_rows(zb, RZC, L, 0.0)
    for k in range(RPS // RZC):
        pltpu.sync_copy(zb, dacc.at[pl.ds(s * RPS + k * RZC, RZC)])
    pltpu.sync_copy(dst3_hbm.at[wid], dstv)
    plsc.subcore_barrier()

    def ring_body(kk, carry):
        for b in range(NDSEM):
            i = kk * NDSEM + b

            @pl.when(kk >= 1)
            def _():
                pltpu.make_async_copy(
                    onesb, dacc.at[dstv.at[i - NDSEM]], ssems[b]).wait()

            pltpu.async_copy(onesb, dacc.at[dstv.at[i]], ssems[b], add=True)
        return carry

    lax.fori_loop(0, DECN // NDSEM, ring_body, 0)
    for b in range(NDSEM):
        i = DECN - NDSEM + b
        pltpu.make_async_copy(onesb, dacc.at[dstv.at[i]], ssems[b]).wait()

    plsc.subcore_barrier()

    r0 = s * RPS
    pltpu.sync_copy(dacc.at[pl.ds(r0, RPS)], degp_hbm.at[c, pl.ds(r0, RPS)])


_deg = pl.kernel(
    _deg_body,
    out_type=jax.ShapeDtypeStruct((NC, Np, L), _f32),
    mesh=_MESH,
    scratch_types=[
        pltpu.VMEM_SHARED((Np, L), _f32),
        pltpu.VMEM((DECN, DEC), jnp.int32),
        pltpu.VMEM((DEC, L), _f32),
        pltpu.VMEM((RZC, L), _f32),
        [pltpu.SemaphoreType.DMA for _ in range(NDSEM)],
    ],
)


NBUF = 5


def _scatter_body(g_hbm, src_hbm, dst_hbm, accp_hbm,
                  acc, srcv, idxd, rows, gsems, ssems, dsems, zsem):
    c = lax.axis_index("c")
    s = lax.axis_index("s")
    wid = c * NS + s
    ebase = wid * EPW

    nz = RPS // EC

    @pl.when(c == 0)
    def _():
        pltpu.async_copy(g_hbm.at[pl.ds(s * RPS, RPS)],
                         acc.at[pl.ds(s * RPS, RPS)], zsem)

    @pl.when(c != 0)
    def _():
        _fill_rows(rows[0], EC, D, 0.0)
        for k in range(nz):
            pltpu.async_copy(rows[0], acc.at[pl.ds(s * RPS + k * EC, EC)],
                             zsem)

    pltpu.sync_copy(src_hbm.at[pl.ds(ebase, EPW)], srcv)

    @pl.when(c == 0)
    def _():
        pltpu.make_async_copy(g_hbm.at[pl.ds(s * RPS, RPS)],
                              acc.at[pl.ds(s * RPS, RPS)], zsem).wait()

    @pl.when(c != 0)
    def _():
        for k in range(nz):
            pltpu.make_async_copy(
                rows[0], acc.at[pl.ds(s * RPS + k * EC, EC)], zsem).wait()

    plsc.subcore_barrier()

    def start_gather(i, b):
        pltpu.async_copy(dst_hbm.at[pl.ds(ebase + i * EC, EC)],
                         idxd[b], dsems[b])
        pltpu.async_copy(g_hbm.at[srcv.at[pl.ds(i * EC, EC)]],
                         rows[b], gsems[b])

    def wait_gather(i, b):
        pltpu.make_async_copy(g_hbm.at[srcv.at[pl.ds(i * EC, EC)]],
                              rows[b], gsems[b]).wait()

    def start_scatter(i, b):
        pltpu.make_async_copy(dst_hbm.at[pl.ds(ebase + i * EC, EC)],
                              idxd[b], dsems[b]).wait()
        pltpu.async_copy(rows[b], acc.at[idxd[b]], ssems[b], add=True)

    def wait_scatter(b):
        pltpu.make_async_copy(rows[b], acc.at[idxd[b]], ssems[b]).wait()

    for b in range(NBUF - 1):
        start_gather(b, b)

    def ring_body(kk, carry):
        for b in range(NBUF):
            i = kk * NBUF + b
            wait_gather(i, b)
            start_scatter(i, b)
            bn = (b + NBUF - 1) % NBUF

            @pl.when(i + NBUF - 1 < ECN)
            def _():
                if b == 0:
                    @pl.when(kk >= 1)
                    def _():
                        wait_scatter(bn)
                else:
                    wait_scatter(bn)
                start_gather(i + NBUF - 1, bn)

        return carry

    lax.fori_loop(0, ECN // NBUF, ring_body, 0)
    for b in range(NBUF):
        wait_scatter(b)
    plsc.subcore_barrier()

    r0 = s * RPS
    pltpu.sync_copy(acc.at[pl.ds(r0, RPS)], accp_hbm.at[c, pl.ds(r0, RPS)])


_scatter = pl.kernel(
    _scatter_body,
    out_type=jax.ShapeDtypeStruct((NC, Np, D), _f32),
    mesh=_MESH,
    scratch_types=[
        pltpu.VMEM_SHARED((Np, D), _f32),
        pltpu.VMEM((EPW,), jnp.int32),
        [pltpu.VMEM((EC,), jnp.int32) for _ in range(NBUF)],
        [pltpu.VMEM((EC, D), _f32) for _ in range(NBUF)],
        [pltpu.SemaphoreType.DMA for _ in range(NBUF)],
        [pltpu.SemaphoreType.DMA for _ in range(NBUF)],
        [pltpu.SemaphoreType.DMA for _ in range(NBUF)],
        pltpu.SemaphoreType.DMA,
    ],
)


RB = 1280


def _prep_body(x_ref, degp_ref, dinv_ref, g_ref):
    deg = 1.0 + degp_ref[0, :, 0:1] + degp_ref[1, :, 0:1]
    dinv = lax.rsqrt(deg)
    dinv_ref[...] = dinv
    g_ref[...] = x_ref[...] * dinv


_prep = pl.pallas_call(
    _prep_body,
    out_shape=(
        jax.ShapeDtypeStruct((Np, 1), _f32),
        jax.ShapeDtypeStruct((Np, D), _f32),
    ),
)


def _dense_body(accp_ref, dinv_ref, w_ref, b_ref, gn_ref):
    u = (accp_ref[0] + accp_ref[1]) * dinv_ref[...]
    h = jnp.dot(u, w_ref[...], preferred_element_type=_f32) + b_ref[...]
    h = jnp.maximum(h, 0.0)
    gn_ref[...] = h * dinv_ref[...]


_dense = pl.pallas_call(
    _dense_body,
    out_shape=jax.ShapeDtypeStruct((Np, D), _f32),
)


def _head_body(accp_ref, dinv_ref, w_ref, b_ref,
               batch_ref, tab_ref, wt_ref, bt_ref,
               w1a_ref, w1b_ref, bf1_ref, w2_ref, bf2_ref,
               w3_ref, bf3_ref, out_ref):
    u = (accp_ref[0] + accp_ref[1]) * dinv_ref[...]
    h3 = jnp.dot(u, w_ref[...], preferred_element_type=_f32) + b_ref[...]
    h3 = jnp.maximum(h3, 0.0)
    gids = lax.broadcasted_iota(jnp.int32, (1, B), 1)
    pmat = (batch_ref[...] == gids).astype(_f32)
    dn = (((0,), (0,)), ((), ()))
    psum = lax.dot_general(pmat, h3, dn,
                           preferred_element_type=_f32)
    cnt = lax.dot_general(pmat, jnp.ones((Np, 1), _f32), dn,
                          preferred_element_type=_f32)
    pooled = psum / jnp.maximum(cnt, 1.0)
    t = jnp.dot(tab_ref[...], wt_ref[...], preferred_element_type=_f32)
    t = jnp.maximum(t + bt_ref[...], 0.0)
    h1 = (jnp.dot(pooled, w1a_ref[...], preferred_element_type=_f32)
          + jnp.dot(t, w1b_ref[...], preferred_element_type=_f32)
          + bf1_ref[...])
    h1 = jnp.maximum(h1, 0.0)
    h2 = jnp.dot(h1, w2_ref[...], preferred_element_type=_f32) + bf2_ref[...]
    h2 = jnp.maximum(h2, 0.0)
    out_ref[...] = jnp.dot(h2, w3_ref[...], preferred_element_type=_f32) + bf3_ref[...]


_head = pl.pallas_call(
    _head_body,
    out_shape=jax.ShapeDtypeStruct((B, 1), _f32),
)


def kernel(x, edge_index, batch, tab_features,
           W1, b1, W2, b2, W3, b3, Wt, bt,
           Wf1, bf1, Wf2, bf2, Wf3, bf3):
    src = edge_index[0].astype(jnp.int32)
    dst = edge_index[1].astype(jnp.int32)
    dst3d = dst.reshape(NW, DECN, DEC)
    batch = batch.astype(jnp.int32)
    xp = jnp.pad(x, ((0, Np - N), (0, 0)))
    batchp = jnp.pad(batch, (0, Np - N), constant_values=B)

    degp = _deg(dst3d)
    dinv, g = _prep(xp, degp)

    for w, b in ((W1, b1), (W2, b2)):
        accp = _scatter(g, src, dst)
        g = _dense(accp, dinv, w, b.reshape(1, D))

    accp = _scatter(g, src, dst)
    out = _head(accp, dinv, W3, b3.reshape(1, D),
                batchp.reshape(Np, 1), tab_features,
                Wt, bt.reshape(1, -1),
                Wf1[:D], Wf1[D:], bf1.reshape(1, -1),
                Wf2, bf2.reshape(1, -1),
                Wf3, bf3.reshape(1, -1))
    return out[:, 0]


# speedup vs baseline: 1.0165x; 1.0165x over previous
import functools

import jax
import jax.numpy as jnp
from jax import lax
from jax.experimental import pallas as pl
from jax.experimental.pallas import tpu as pltpu
from jax.experimental.pallas import tpu_sc as plsc

N = 10000
Np = 10240
E = 320000
D = 128
B = 512
NC, NS, L = 2, 16, 16
NW = NC * NS

EPW = E // NW
EC = 40
ECN = EPW // EC
DEC = 80
DECN = EPW // DEC
RPS = Np // NS
RZC = 160
PC = 40
PCN = (N + PC - 1) // PC
PCI = (PCN + NW - 1) // NW

_MESH = plsc.VectorSubcoreMesh(
    core_axis_name="c", subcore_axis_name="s", num_cores=NC, num_subcores=NS
)

_f32 = jnp.float32


def _fill_rows(buf, nrows, ncols, value):
    vec = jnp.full((L,), value, _f32)

    def body(i, carry):
        for k in range(ncols // L):
            buf[i, pl.ds(k * L, L)] = vec
        return carry

    lax.fori_loop(0, nrows, body, 0)


NDSEM = 5


def _deg_body(dst3_hbm, degp_hbm, dacc, dstv, onesb, zb, ssems):
    c = lax.axis_index("c")
    s = lax.axis_index("s")
    wid = c * NS + s

    _fill_rows(onesb, DEC, L, 1.0)
    _fill_rows(zb, RZC, L, 0.0)
    for k in range(RPS // RZC):
        pltpu.sync_copy(zb, dacc.at[pl.ds(s * RPS + k * RZC, RZC)])
    pltpu.sync_copy(dst3_hbm.at[wid], dstv)
    plsc.subcore_barrier()

    def ring_body(kk, carry):
        for b in range(NDSEM):
            i = kk * NDSEM + b

            @pl.when(kk >= 1)
            def _():
                pltpu.make_async_copy(
                    onesb, dacc.at[dstv.at[i - NDSEM]], ssems[b]).wait()

            pltpu.async_copy(onesb, dacc.at[dstv.at[i]], ssems[b], add=True)
        return carry

    lax.fori_loop(0, DECN // NDSEM, ring_body, 0)
    for b in range(NDSEM):
        i = DECN - NDSEM + b
        pltpu.make_async_copy(onesb, dacc.at[dstv.at[i]], ssems[b]).wait()

    plsc.subcore_barrier()

    r0 = s * RPS
    pltpu.sync_copy(dacc.at[pl.ds(r0, RPS)], degp_hbm.at[c, pl.ds(r0, RPS)])


_deg = pl.kernel(
    _deg_body,
    out_type=jax.ShapeDtypeStruct((NC, Np, L), _f32),
    mesh=_MESH,
    scratch_types=[
        pltpu.VMEM_SHARED((Np, L), _f32),
        pltpu.VMEM((DECN, DEC), jnp.int32),
        pltpu.VMEM((DEC, L), _f32),
        pltpu.VMEM((RZC, L), _f32),
        [pltpu.SemaphoreType.DMA for _ in range(NDSEM)],
    ],
)


NBUF = 5


def _scatter_body(g_hbm, src_hbm, dst_hbm, accp_hbm,
                  acc, srcv, idxd, rows, gsems, ssems, dsems, zsem):
    c = lax.axis_index("c")
    s = lax.axis_index("s")
    wid = c * NS + s
    ebase = wid * EPW

    _fill_rows(rows[0], EC, D, 0.0)
    nz = RPS // EC
    for k in range(nz):
        pltpu.async_copy(rows[0], acc.at[pl.ds(s * RPS + k * EC, EC)], zsem)
    pltpu.sync_copy(src_hbm.at[pl.ds(ebase, EPW)], srcv)
    for k in range(nz):
        pltpu.make_async_copy(
            rows[0], acc.at[pl.ds(s * RPS + k * EC, EC)], zsem).wait()
    plsc.subcore_barrier()

    def start_gather(i, b):
        pltpu.async_copy(dst_hbm.at[pl.ds(ebase + i * EC, EC)],
                         idxd[b], dsems[b])
        pltpu.async_copy(g_hbm.at[srcv.at[pl.ds(i * EC, EC)]],
                         rows[b], gsems[b])

    def wait_gather(i, b):
        pltpu.make_async_copy(g_hbm.at[srcv.at[pl.ds(i * EC, EC)]],
                              rows[b], gsems[b]).wait()

    def start_scatter(i, b):
        pltpu.make_async_copy(dst_hbm.at[pl.ds(ebase + i * EC, EC)],
                              idxd[b], dsems[b]).wait()
        pltpu.async_copy(rows[b], acc.at[idxd[b]], ssems[b], add=True)

    def wait_scatter(b):
        pltpu.make_async_copy(rows[b], acc.at[idxd[b]], ssems[b]).wait()

    for b in range(NBUF - 1):
        start_gather(b, b)

    def ring_body(kk, carry):
        for b in range(NBUF):
            i = kk * NBUF + b
            wait_gather(i, b)
            start_scatter(i, b)
            bn = (b + NBUF - 1) % NBUF

            @pl.when(i + NBUF - 1 < ECN)
            def _():
                if b == 0:
                    @pl.when(kk >= 1)
                    def _():
                        wait_scatter(bn)
                else:
                    wait_scatter(bn)
                start_gather(i + NBUF - 1, bn)

        return carry

    lax.fori_loop(0, ECN // NBUF, ring_body, 0)
    for b in range(NBUF):
        wait_scatter(b)
    plsc.subcore_barrier()

    r0 = s * RPS
    pltpu.sync_copy(acc.at[pl.ds(r0, RPS)], accp_hbm.at[c, pl.ds(r0, RPS)])


_scatter = pl.kernel(
    _scatter_body,
    out_type=jax.ShapeDtypeStruct((NC, Np, D), _f32),
    mesh=_MESH,
    scratch_types=[
        pltpu.VMEM_SHARED((Np, D), _f32),
        pltpu.VMEM((EPW,), jnp.int32),
        [pltpu.VMEM((EC,), jnp.int32) for _ in range(NBUF)],
        [pltpu.VMEM((EC, D), _f32) for _ in range(NBUF)],
        [pltpu.SemaphoreType.DMA for _ in range(NBUF)],
        [pltpu.SemaphoreType.DMA for _ in range(NBUF)],
        [pltpu.SemaphoreType.DMA for _ in range(NBUF)],
        pltpu.SemaphoreType.DMA,
    ],
)


RB = 1280


def _prep_body(x_ref, degp_ref, dinv_ref, g_ref):
    deg = 1.0 + degp_ref[0, :, 0:1] + degp_ref[1, :, 0:1]
    dinv = lax.rsqrt(deg)
    dinv_ref[...] = dinv
    g_ref[...] = x_ref[...] * dinv


_prep = pl.pallas_call(
    _prep_body,
    out_shape=(
        jax.ShapeDtypeStruct((Np, 1), _f32),
        jax.ShapeDtypeStruct((Np, D), _f32),
    ),
)


def _dense_body(accp_ref, g_ref, dinv_ref, w_ref, b_ref, gn_ref):
    u = (accp_ref[0] + accp_ref[1] + g_ref[...]) * dinv_ref[...]
    h = jnp.dot(u, w_ref[...], preferred_element_type=_f32) + b_ref[...]
    h = jnp.maximum(h, 0.0)
    gn_ref[...] = h * dinv_ref[...]


_dense = pl.pallas_call(
    _dense_body,
    out_shape=jax.ShapeDtypeStruct((Np, D), _f32),
)


def _head_body(accp_ref, g_ref, dinv_ref, w_ref, b_ref,
               batch_ref, tab_ref, wt_ref, bt_ref,
               w1a_ref, w1b_ref, bf1_ref, w2_ref, bf2_ref,
               w3_ref, bf3_ref, out_ref):
    u = (accp_ref[0] + accp_ref[1] + g_ref[...]) * dinv_ref[...]
    h3 = jnp.dot(u, w_ref[...], preferred_element_type=_f32) + b_ref[...]
    h3 = jnp.maximum(h3, 0.0)
    gids = lax.broadcasted_iota(jnp.int32, (1, B), 1)
    pmat = (batch_ref[...] == gids).astype(_f32)
    dn = (((0,), (0,)), ((), ()))
    psum = lax.dot_general(pmat, h3, dn,
                           preferred_element_type=_f32)
    cnt = lax.dot_general(pmat, jnp.ones((Np, 1), _f32), dn,
                          preferred_element_type=_f32)
    pooled = psum / jnp.maximum(cnt, 1.0)
    t = jnp.dot(tab_ref[...], wt_ref[...], preferred_element_type=_f32)
    t = jnp.maximum(t + bt_ref[...], 0.0)
    h1 = (jnp.dot(pooled, w1a_ref[...], preferred_element_type=_f32)
          + jnp.dot(t, w1b_ref[...], preferred_element_type=_f32)
          + bf1_ref[...])
    h1 = jnp.maximum(h1, 0.0)
    h2 = jnp.dot(h1, w2_ref[...], preferred_element_type=_f32) + bf2_ref[...]
    h2 = jnp.maximum(h2, 0.0)
    out_ref[...] = jnp.dot(h2, w3_ref[...], preferred_element_type=_f32) + bf3_ref[...]


_head = pl.pallas_call(
    _head_body,
    out_shape=jax.ShapeDtypeStruct((B, 1), _f32),
)


def kernel(x, edge_index, batch, tab_features,
           W1, b1, W2, b2, W3, b3, Wt, bt,
           Wf1, bf1, Wf2, bf2, Wf3, bf3):
    src = edge_index[0].astype(jnp.int32)
    dst = edge_index[1].astype(jnp.int32)
    dst3d = dst.reshape(NW, DECN, DEC)
    batch = batch.astype(jnp.int32)
    xp = jnp.pad(x, ((0, Np - N), (0, 0)))
    batchp = jnp.pad(batch, (0, Np - N), constant_values=B)

    degp = _deg(dst3d)
    dinv, g = _prep(xp, degp)

    for w, b in ((W1, b1), (W2, b2)):
        accp = _scatter(g, src, dst)
        g = _dense(accp, g, dinv, w, b.reshape(1, D))

    accp = _scatter(g, src, dst)
    out = _head(accp, g, dinv, W3, b3.reshape(1, D),
                batchp.reshape(Np, 1), tab_features,
                Wt, bt.reshape(1, -1),
                Wf1[:D], Wf1[D:], bf1.reshape(1, -1),
                Wf2, bf2.reshape(1, -1),
                Wf3, bf3.reshape(1, -1))
    return out[:, 0]


# speedup vs baseline: 1.0175x; 1.0010x over previous
import jax
import jax.numpy as jnp
from jax import lax
from jax.experimental import pallas as pl
from jax.experimental.pallas import tpu as pltpu
from jax.experimental.pallas import tpu_sc as plsc

N = 10000
Np = 10240
E = 320000
D = 128
B = 512
NC, NS, L = 2, 16, 16
NW = NC * NS

EPW = E // NW
EC = 40
ECN = EPW // EC
DEC = 80
DECN = EPW // DEC
RPS = Np // NS
RZC = 160
_MESH = plsc.VectorSubcoreMesh(
    core_axis_name="c", subcore_axis_name="s", num_cores=NC, num_subcores=NS
)

_f32 = jnp.float32


def _fill_rows(buf, nrows, ncols, value):
    vec = jnp.full((L,), value, _f32)

    def body(i, carry):
        for k in range(ncols // L):
            buf[i, pl.ds(k * L, L)] = vec
        return carry

    lax.fori_loop(0, nrows, body, 0)


NDSEM = 5


def _deg_body(dst3_hbm, degp_hbm, dacc, dstv, onesb, zb, ssems):
    c = lax.axis_index("c")
    s = lax.axis_index("s")
    wid = c * NS + s

    _fill_rows(onesb, DEC, L, 1.0)
    _fill_rows(zb, RZC, L, 0.0)
    for k in range(RPS // RZC):
        pltpu.sync_copy(zb, dacc.at[pl.ds(s * RPS + k * RZC, RZC)])
    pltpu.sync_copy(dst3_hbm.at[wid], dstv)
    plsc.subcore_barrier()

    def ring_body(kk, carry):
        for b in range(NDSEM):
            i = kk * NDSEM + b

            @pl.when(kk >= 1)
            def _():
                pltpu.make_async_copy(
                    onesb, dacc.at[dstv.at[i - NDSEM]], ssems[b]).wait()

            pltpu.async_copy(onesb, dacc.at[dstv.at[i]], ssems[b], add=True)
        return carry

    lax.fori_loop(0, DECN // NDSEM, ring_body, 0)
    for b in range(NDSEM):
        i = DECN - NDSEM + b
        pltpu.make_async_copy(onesb, dacc.at[dstv.at[i]], ssems[b]).wait()

    plsc.subcore_barrier()

    r0 = s * RPS
    pltpu.sync_copy(dacc.at[pl.ds(r0, RPS)], degp_hbm.at[c, pl.ds(r0, RPS)])


_deg = pl.kernel(
    _deg_body,
    out_type=jax.ShapeDtypeStruct((NC, Np, L), _f32),
    mesh=_MESH,
    scratch_types=[
        pltpu.VMEM_SHARED((Np, L), _f32),
        pltpu.VMEM((DECN, DEC), jnp.int32),
        pltpu.VMEM((DEC, L), _f32),
        pltpu.VMEM((RZC, L), _f32),
        [pltpu.SemaphoreType.DMA for _ in range(NDSEM)],
    ],
)


NBUF = 5


def _scatter_body(g_hbm, src_hbm, dst_hbm, accp_hbm,
                  acc, srcv, idxd, rows, gsems, ssems, dsems, zsem):
    c = lax.axis_index("c")
    s = lax.axis_index("s")
    wid = c * NS + s
    ebase = wid * EPW

    _fill_rows(rows[0], EC, D, 0.0)
    nz = RPS // EC
    for k in range(nz):
        pltpu.async_copy(rows[0], acc.at[pl.ds(s * RPS + k * EC, EC)], zsem)
    pltpu.sync_copy(src_hbm.at[pl.ds(ebase, EPW)], srcv)
    for k in range(nz):
        pltpu.make_async_copy(
            rows[0], acc.at[pl.ds(s * RPS + k * EC, EC)], zsem).wait()
    plsc.subcore_barrier()

    def start_gather(i, b):
        pltpu.async_copy(dst_hbm.at[pl.ds(ebase + i * EC, EC)],
                         idxd[b], dsems[b])
        pltpu.async_copy(g_hbm.at[srcv.at[pl.ds(i * EC, EC)]],
                         rows[b], gsems[b])

    def wait_gather(i, b):
        pltpu.make_async_copy(g_hbm.at[srcv.at[pl.ds(i * EC, EC)]],
                              rows[b], gsems[b]).wait()

    def start_scatter(i, b):
        pltpu.make_async_copy(dst_hbm.at[pl.ds(ebase + i * EC, EC)],
                              idxd[b], dsems[b]).wait()
        pltpu.async_copy(rows[b], acc.at[idxd[b]], ssems[b], add=True)

    def wait_scatter(b):
        pltpu.make_async_copy(rows[b], acc.at[idxd[b]], ssems[b]).wait()

    for b in range(NBUF - 1):
        start_gather(b, b)

    def ring_body(kk, carry):
        for b in range(NBUF):
            i = kk * NBUF + b
            wait_gather(i, b)
            start_scatter(i, b)
            bn = (b + NBUF - 1) % NBUF

            @pl.when(i + NBUF - 1 < ECN)
            def _():
                if b == 0:
                    @pl.when(kk >= 1)
                    def _():
                        wait_scatter(bn)
                else:
                    wait_scatter(bn)
                start_gather(i + NBUF - 1, bn)

        return carry

    lax.fori_loop(0, ECN // NBUF, ring_body, 0)
    for b in range(NBUF):
        wait_scatter(b)
    plsc.subcore_barrier()

    r0 = s * RPS
    pltpu.sync_copy(acc.at[pl.ds(r0, RPS)], accp_hbm.at[c, pl.ds(r0, RPS)])


_scatter = pl.kernel(
    _scatter_body,
    out_type=jax.ShapeDtypeStruct((NC, Np, D), _f32),
    mesh=_MESH,
    scratch_types=[
        pltpu.VMEM_SHARED((Np, D), _f32),
        pltpu.VMEM((EPW,), jnp.int32),
        [pltpu.VMEM((EC,), jnp.int32) for _ in range(NBUF)],
        [pltpu.VMEM((EC, D), _f32) for _ in range(NBUF)],
        [pltpu.SemaphoreType.DMA for _ in range(NBUF)],
        [pltpu.SemaphoreType.DMA for _ in range(NBUF)],
        [pltpu.SemaphoreType.DMA for _ in range(NBUF)],
        pltpu.SemaphoreType.DMA,
    ],
)


RB = 1280


def _prep_body(x_ref, degp_ref, dinv_ref, g_ref):
    deg = 1.0 + degp_ref[0, :, 0:1] + degp_ref[1, :, 0:1]
    dinv = lax.rsqrt(deg)
    dinv_ref[...] = dinv
    g_ref[...] = x_ref[...] * dinv


_prep = pl.pallas_call(
    _prep_body,
    out_shape=(
        jax.ShapeDtypeStruct((Np, 1), _f32),
        jax.ShapeDtypeStruct((Np, D), _f32),
    ),
)


def _dense_body(accp_ref, g_ref, dinv_ref, w_ref, b_ref, gn_ref):
    u = (accp_ref[0] + accp_ref[1] + g_ref[...]) * dinv_ref[...]
    h = jnp.dot(u, w_ref[...], preferred_element_type=_f32) + b_ref[...]
    h = jnp.maximum(h, 0.0)
    gn_ref[...] = h * dinv_ref[...]


_dense = pl.pallas_call(
    _dense_body,
    out_shape=jax.ShapeDtypeStruct((Np, D), _f32),
)


def _head_body(accp_ref, g_ref, dinv_ref, w_ref, b_ref,
               batch_ref, tab_ref, wt_ref, bt_ref,
               w1a_ref, w1b_ref, bf1_ref, w2_ref, bf2_ref,
               w3_ref, bf3_ref, out_ref):
    u = (accp_ref[0] + accp_ref[1] + g_ref[...]) * dinv_ref[...]
    h3 = jnp.dot(u, w_ref[...], preferred_element_type=_f32) + b_ref[...]
    h3 = jnp.maximum(h3, 0.0)
    gids = lax.broadcasted_iota(jnp.int32, (1, B), 1)
    pmat = (batch_ref[...] == gids).astype(_f32)
    dn = (((0,), (0,)), ((), ()))
    psum = lax.dot_general(pmat, h3, dn,
                           preferred_element_type=_f32)
    cnt = lax.dot_general(pmat, jnp.ones((Np, 1), _f32), dn,
                          preferred_element_type=_f32)
    pooled = psum / jnp.maximum(cnt, 1.0)
    t = jnp.dot(tab_ref[...], wt_ref[...], preferred_element_type=_f32)
    t = jnp.maximum(t + bt_ref[...], 0.0)
    h1 = (jnp.dot(pooled, w1a_ref[...], preferred_element_type=_f32)
          + jnp.dot(t, w1b_ref[...], preferred_element_type=_f32)
          + bf1_ref[...])
    h1 = jnp.maximum(h1, 0.0)
    h2 = jnp.dot(h1, w2_ref[...], preferred_element_type=_f32) + bf2_ref[...]
    h2 = jnp.maximum(h2, 0.0)
    out_ref[...] = jnp.dot(h2, w3_ref[...], preferred_element_type=_f32) + bf3_ref[...]


_head = pl.pallas_call(
    _head_body,
    out_shape=jax.ShapeDtypeStruct((B, 1), _f32),
)


def kernel(x, edge_index, batch, tab_features,
           W1, b1, W2, b2, W3, b3, Wt, bt,
           Wf1, bf1, Wf2, bf2, Wf3, bf3):
    src = edge_index[0].astype(jnp.int32)
    dst = edge_index[1].astype(jnp.int32)
    dst3d = dst.reshape(NW, DECN, DEC)
    batch = batch.astype(jnp.int32)
    xp = jnp.pad(x, ((0, Np - N), (0, 0)))
    batchp = jnp.pad(batch, (0, Np - N), constant_values=B)

    degp = _deg(dst3d)
    dinv, g = _prep(xp, degp)

    for w, b in ((W1, b1), (W2, b2)):
        accp = _scatter(g, src, dst)
        g = _dense(accp, g, dinv, w, b.reshape(1, D))

    accp = _scatter(g, src, dst)
    out = _head(accp, g, dinv, W3, b3.reshape(1, D),
                batchp.reshape(Np, 1), tab_features,
                Wt, bt.reshape(1, -1),
                Wf1[:D], Wf1[D:], bf1.reshape(1, -1),
                Wf2, bf2.reshape(1, -1),
                Wf3, bf3.reshape(1, -1))
    return out[:, 0]
